# baseline (speedup 1.0000x reference)
import functools

import jax
import jax.numpy as jnp
from jax import lax
from jax.experimental import pallas as pl
from jax.experimental.pallas import tpu as pltpu
from jax.experimental.pallas import tpu_sc as plsc

BATCH = 16384
EMBED_DIM = 64
_PACK = 2 * EMBED_DIM

_NC, _NS = 2, 16
_NW = _NC * _NS
_B_PER_W = BATCH // _NW
_LANES = 16
_CHUNK = 256
_NCHUNK = _B_PER_W // _CHUNK


_REPACK_VC = 8192


def _bf16_bits(y):
    u = lax.bitcast_convert_type(y, jnp.uint32)
    return (u + 0x7FFF + ((u >> 16) & 1)) >> 16


def _repack_body(x_ref, o_ref):
    q = _REPACK_VC // 4
    x = x_ref[...]
    a = [_bf16_bits(x[:, k * q:(k + 1) * q].T) for k in range(4)]
    lo = a[0] | (a[1] << 16)
    hi = a[2] | (a[3] << 16)
    o_ref[...] = jnp.concatenate([lo, hi], axis=1)


@functools.partial(jax.jit, static_argnames=("nblk",))
def _tc_repack(table_t, nblk):
    vc = _REPACK_VC
    return pl.pallas_call(
        _repack_body,
        grid=(nblk,),
        in_specs=[pl.BlockSpec((EMBED_DIM, vc), lambda i: (0, i))],
        out_specs=pl.BlockSpec((vc // 4, 2 * EMBED_DIM), lambda i: (i, 0)),
        out_shape=jax.ShapeDtypeStruct((nblk * (vc // 4), 2 * EMBED_DIM),
                                       jnp.uint32),
    )(table_t)


def _extract(vec, j):
    lane = lax.broadcasted_iota(jnp.int32, (_LANES,), 0)
    masked = jnp.where(lane == j, vec, jnp.int32(0))
    return jnp.max(masked)


def _gather_body(lt_hbm, ct_hbm, li_hbm, ci_hbm, el_hbm, ec_hbm,
                 li_v, ci_v, el_v, ec_v, sem):
    wid = lax.axis_index("s") * _NC + lax.axis_index("c")
    base = wid * _B_PER_W
    pltpu.sync_copy(li_hbm.at[pl.ds(base, _B_PER_W)], li_v)
    pltpu.sync_copy(ci_hbm.at[pl.ds(base, _B_PER_W)], ci_v)
    for c in range(_NCHUNK):
        off = c * _CHUNK

        def issue(g, _):
            lv = li_v[pl.ds(off + g * _LANES, _LANES)]
            cv = ci_v[pl.ds(off + g * _LANES, _LANES)]
            for j in range(_LANES):
                li = _extract(lv, j)
                ci = _extract(cv, j)
                i = g * _LANES + j
                pltpu.make_async_copy(lt_hbm.at[pl.ds(li, 1), :],
                                      el_v.at[pl.ds(i, 1), :], sem).start()
                pltpu.make_async_copy(ct_hbm.at[pl.ds(ci, 1), :],
                                      ec_v.at[pl.ds(i, 1), :], sem).start()
            return 0

        lax.fori_loop(0, _CHUNK // _LANES, issue, 0)
        pltpu.make_async_copy(lt_hbm.at[pl.ds(0, _CHUNK), :], el_v, sem).wait()
        pltpu.make_async_copy(ct_hbm.at[pl.ds(0, _CHUNK), :], ec_v, sem).wait()
        pltpu.sync_copy(el_v, el_hbm.at[pl.ds(base + off, _CHUNK)])
        pltpu.sync_copy(ec_v, ec_hbm.at[pl.ds(base + off, _CHUNK)])


@jax.jit
def _sc_gather(lt_packed, ct_packed, li_half, ci_half):
    mesh = plsc.VectorSubcoreMesh(core_axis_name="c", subcore_axis_name="s")
    out_type = [
        jax.ShapeDtypeStruct((BATCH, _PACK), jnp.uint32),
        jax.ShapeDtypeStruct((BATCH, _PACK), jnp.uint32),
    ]
    scratch = [
        pltpu.VMEM((_B_PER_W,), jnp.int32),
        pltpu.VMEM((_B_PER_W,), jnp.int32),
        pltpu.VMEM((_CHUNK, _PACK), jnp.uint32),
        pltpu.VMEM((_CHUNK, _PACK), jnp.uint32),
        pltpu.SemaphoreType.DMA,
    ]
    fn = pl.kernel(_gather_body, out_type=out_type, mesh=mesh,
                   scratch_types=scratch,
                   compiler_params=pltpu.CompilerParams(
                       needs_layout_passes=False))
    return fn(lt_packed, ct_packed, li_half, ci_half)


def _unpack_select(xu, p):
    half = jnp.where(p >= 2, xu[:, EMBED_DIM:], xu[:, :EMBED_DIM])
    bits = jnp.where((p & 1) == 1, half >> 16, half & 0xFFFF)
    return lax.bitcast_convert_type(bits << 16, jnp.float32)


def _mlp_body(x1_ref, x2_ref, p1_ref, p2_ref, w1a_ref, w1b_ref, b1_ref,
              w2_ref, b2_ref, w3_ref, b3_ref, o_ref):
    x1 = _unpack_select(x1_ref[...], p1_ref[...])
    x2 = _unpack_select(x2_ref[...], p2_ref[...])
    h = jnp.dot(x1, w1a_ref[...], preferred_element_type=jnp.float32)
    h += jnp.dot(x2, w1b_ref[...], preferred_element_type=jnp.float32)
    h = jnp.maximum(h + b1_ref[...], 0.0)
    h = jnp.dot(h, w2_ref[...], preferred_element_type=jnp.float32)
    h = jnp.maximum(h + b2_ref[...], 0.0)
    o_ref[...] = jnp.dot(h, w3_ref[...],
                         preferred_element_type=jnp.float32) + b3_ref[...]


@functools.partial(jax.jit, static_argnames=("bm",))
def _tc_mlp(e_label, e_cat, p1, p2, W1a, W1b, b1, W2, b2, W3, b3, bm=2048):
    grid = (BATCH // bm,)
    full = lambda shape: pl.BlockSpec(shape, lambda i: (0, 0))
    return pl.pallas_call(
        _mlp_body,
        grid=grid,
        in_specs=[
            pl.BlockSpec((bm, _PACK), lambda i: (i, 0)),
            pl.BlockSpec((bm, _PACK), lambda i: (i, 0)),
            pl.BlockSpec((bm, 1), lambda i: (i, 0)),
            pl.BlockSpec((bm, 1), lambda i: (i, 0)),
            full(W1a.shape),
            full(W1b.shape),
            full(b1.shape),
            full(W2.shape),
            full(b2.shape),
            full(W3.shape),
            full(b3.shape),
        ],
        out_specs=pl.BlockSpec((bm, 2), lambda i: (i, 0)),
        out_shape=jax.ShapeDtypeStruct((BATCH, 2), jnp.float32),
    )(e_label, e_cat, p1, p2, W1a, W1b, b1, W2, b2, W3, b3)


def kernel(label_idx, category_idx, label_table, cat_table,
           W1, b1, W2, b2, W3, b3):
    li = label_idx.astype(jnp.int32)
    ci = category_idx.astype(jnp.int32)
    vc = _REPACK_VC
    nblk_l = -(-1000000 // vc)
    nblk_c = -(-100000 // vc)
    lt_packed = _tc_repack(label_table.T, nblk_l)
    ct_packed = _tc_repack(cat_table.T, nblk_c)
    qm = vc // 4 - 1
    lrow = ((li >> 13) << 11) + (li & qm)
    crow = ((ci >> 13) << 11) + (ci & qm)
    e_label, e_cat = _sc_gather(lt_packed, ct_packed, lrow, crow)
    p1 = ((li >> 11) & 3).reshape(-1, 1)
    p2 = ((ci >> 11) & 3).reshape(-1, 1)
    W1a = W1[:EMBED_DIM]
    W1b = W1[EMBED_DIM:]
    return _tc_mlp(e_label, e_cat, p1, p2, W1a, W1b, b1.reshape(1, -1), W2,
                   b2.reshape(1, -1), W3, b3.reshape(1, -1))


# speedup vs baseline: 1.1280x; 1.1280x over previous
import functools

import jax
import jax.numpy as jnp
from jax import lax
from jax.experimental import pallas as pl
from jax.experimental.pallas import tpu as pltpu
from jax.experimental.pallas import tpu_sc as plsc

BATCH = 16384
EMBED_DIM = 64
_PACK = 2 * EMBED_DIM

_NC, _NS = 2, 16
_NW = _NC * _NS
_B_PER_W = BATCH // _NW
_LANES = 16
_CHUNK = 256
_NCHUNK = _B_PER_W // _CHUNK


_REPACK_VC = 16384


def _bf16_bits(y):
    u = lax.bitcast_convert_type(y, jnp.uint32)
    return (u + 0x7FFF + ((u >> 16) & 1)) >> 16


def _repack_body(x_ref, o_ref):
    q = _REPACK_VC // 4
    x = x_ref[...]
    a = [_bf16_bits(x[:, k * q:(k + 1) * q].T) for k in range(4)]
    lo = a[0] | (a[1] << 16)
    hi = a[2] | (a[3] << 16)
    o_ref[...] = jnp.concatenate([lo, hi], axis=1)


@functools.partial(jax.jit, static_argnames=("nblk",))
def _tc_repack(table_t, nblk):
    vc = _REPACK_VC
    return pl.pallas_call(
        _repack_body,
        grid=(nblk,),
        in_specs=[pl.BlockSpec((EMBED_DIM, vc), lambda i: (0, i))],
        out_specs=pl.BlockSpec((vc // 4, 2 * EMBED_DIM), lambda i: (i, 0)),
        out_shape=jax.ShapeDtypeStruct((nblk * (vc // 4), 2 * EMBED_DIM),
                                       jnp.uint32),
    )(table_t)


def _extract(vec, j):
    lane = lax.broadcasted_iota(jnp.int32, (_LANES,), 0)
    masked = jnp.where(lane == j, vec, jnp.int32(0))
    return jnp.max(masked)


def _gather_body(lt_hbm, ct_hbm, li_hbm, ci_hbm, el_hbm, ec_hbm,
                 li_v, ci_v, el_v, ec_v, sem):
    wid = lax.axis_index("s") * _NC + lax.axis_index("c")
    base = wid * _B_PER_W
    pltpu.sync_copy(li_hbm.at[pl.ds(base, _B_PER_W)], li_v)
    pltpu.sync_copy(ci_hbm.at[pl.ds(base, _B_PER_W)], ci_v)
    for c in range(_NCHUNK):
        off = c * _CHUNK

        def issue(g, _):
            lv = li_v[pl.ds(off + g * _LANES, _LANES)]
            cv = ci_v[pl.ds(off + g * _LANES, _LANES)]
            for j in range(_LANES):
                li = _extract(lv, j)
                ci = _extract(cv, j)
                i = g * _LANES + j
                pltpu.make_async_copy(lt_hbm.at[pl.ds(li, 1), :],
                                      el_v.at[pl.ds(i, 1), :], sem).start()
                pltpu.make_async_copy(ct_hbm.at[pl.ds(ci, 1), :],
                                      ec_v.at[pl.ds(i, 1), :], sem).start()
            return 0

        lax.fori_loop(0, _CHUNK // _LANES, issue, 0)
        pltpu.make_async_copy(lt_hbm.at[pl.ds(0, _CHUNK), :], el_v, sem).wait()
        pltpu.make_async_copy(ct_hbm.at[pl.ds(0, _CHUNK), :], ec_v, sem).wait()
        pltpu.sync_copy(el_v, el_hbm.at[pl.ds(base + off, _CHUNK)])
        pltpu.sync_copy(ec_v, ec_hbm.at[pl.ds(base + off, _CHUNK)])


@jax.jit
def _sc_gather(lt_packed, ct_packed, li_half, ci_half):
    mesh = plsc.VectorSubcoreMesh(core_axis_name="c", subcore_axis_name="s")
    out_type = [
        jax.ShapeDtypeStruct((BATCH, _PACK), jnp.uint32),
        jax.ShapeDtypeStruct((BATCH, _PACK), jnp.uint32),
    ]
    scratch = [
        pltpu.VMEM((_B_PER_W,), jnp.int32),
        pltpu.VMEM((_B_PER_W,), jnp.int32),
        pltpu.VMEM((_CHUNK, _PACK), jnp.uint32),
        pltpu.VMEM((_CHUNK, _PACK), jnp.uint32),
        pltpu.SemaphoreType.DMA,
    ]
    fn = pl.kernel(_gather_body, out_type=out_type, mesh=mesh,
                   scratch_types=scratch,
                   compiler_params=pltpu.CompilerParams(
                       needs_layout_passes=False))
    return fn(lt_packed, ct_packed, li_half, ci_half)


def _unpack_select(xu, p):
    half = jnp.where(p >= 2, xu[:, EMBED_DIM:], xu[:, :EMBED_DIM])
    bits = jnp.where((p & 1) == 1, half >> 16, half & 0xFFFF)
    return lax.bitcast_convert_type(bits << 16, jnp.float32)


def _mlp_body(x1_ref, x2_ref, p1_ref, p2_ref, w1a_ref, w1b_ref, b1_ref,
              w2_ref, b2_ref, w3_ref, b3_ref, o_ref):
    x1 = _unpack_select(x1_ref[...], p1_ref[...])
    x2 = _unpack_select(x2_ref[...], p2_ref[...])
    h = jnp.dot(x1, w1a_ref[...], preferred_element_type=jnp.float32)
    h += jnp.dot(x2, w1b_ref[...], preferred_element_type=jnp.float32)
    h = jnp.maximum(h + b1_ref[...], 0.0)
    h = jnp.dot(h, w2_ref[...], preferred_element_type=jnp.float32)
    h = jnp.maximum(h + b2_ref[...], 0.0)
    o_ref[...] = jnp.dot(h, w3_ref[...],
                         preferred_element_type=jnp.float32) + b3_ref[...]


@functools.partial(jax.jit, static_argnames=("bm",))
def _tc_mlp(e_label, e_cat, p1, p2, W1a, W1b, b1, W2, b2, W3, b3, bm=2048):
    grid = (BATCH // bm,)
    full = lambda shape: pl.BlockSpec(shape, lambda i: (0, 0))
    return pl.pallas_call(
        _mlp_body,
        grid=grid,
        in_specs=[
            pl.BlockSpec((bm, _PACK), lambda i: (i, 0)),
            pl.BlockSpec((bm, _PACK), lambda i: (i, 0)),
            pl.BlockSpec((bm, 1), lambda i: (i, 0)),
            pl.BlockSpec((bm, 1), lambda i: (i, 0)),
            full(W1a.shape),
            full(W1b.shape),
            full(b1.shape),
            full(W2.shape),
            full(b2.shape),
            full(W3.shape),
            full(b3.shape),
        ],
        out_specs=pl.BlockSpec((bm, 2), lambda i: (i, 0)),
        out_shape=jax.ShapeDtypeStruct((BATCH, 2), jnp.float32),
    )(e_label, e_cat, p1, p2, W1a, W1b, b1, W2, b2, W3, b3)


def kernel(label_idx, category_idx, label_table, cat_table,
           W1, b1, W2, b2, W3, b3):
    li = label_idx.astype(jnp.int32)
    ci = category_idx.astype(jnp.int32)
    vc = _REPACK_VC
    nblk_l = -(-1000000 // vc)
    nblk_c = -(-100000 // vc)
    lt_packed = _tc_repack(label_table.T, nblk_l)
    ct_packed = _tc_repack(cat_table.T, nblk_c)
    qm = vc // 4 - 1
    lrow = ((li >> 14) << 12) + (li & qm)
    crow = ((ci >> 14) << 12) + (ci & qm)
    e_label, e_cat = _sc_gather(lt_packed, ct_packed, lrow, crow)
    p1 = ((li >> 12) & 3).reshape(-1, 1)
    p2 = ((ci >> 12) & 3).reshape(-1, 1)
    W1a = W1[:EMBED_DIM]
    W1b = W1[EMBED_DIM:]
    return _tc_mlp(e_label, e_cat, p1, p2, W1a, W1b, b1.reshape(1, -1), W2,
                   b2.reshape(1, -1), W3, b3.reshape(1, -1))


# speedup vs baseline: 1.1378x; 1.0087x over previous
import functools

import jax
import jax.numpy as jnp
from jax import lax
from jax.experimental import pallas as pl
from jax.experimental.pallas import tpu as pltpu
from jax.experimental.pallas import tpu_sc as plsc

BATCH = 16384
EMBED_DIM = 64
_PACK = 2 * EMBED_DIM

_NC, _NS = 2, 16
_NW = _NC * _NS
_B_PER_W = BATCH // _NW
_LANES = 16
_CHUNK = 256
_NCHUNK = _B_PER_W // _CHUNK


_REPACK_VC = 32768


def _bf16_bits(y):
    u = lax.bitcast_convert_type(y, jnp.uint32)
    return (u + 0x7FFF + ((u >> 16) & 1)) >> 16


def _repack_body(x_ref, o_ref):
    q = _REPACK_VC // 4
    x = x_ref[...]
    a = [_bf16_bits(x[:, k * q:(k + 1) * q].T) for k in range(4)]
    lo = a[0] | (a[1] << 16)
    hi = a[2] | (a[3] << 16)
    o_ref[...] = jnp.concatenate([lo, hi], axis=1)


@functools.partial(jax.jit, static_argnames=("nblk",))
def _tc_repack(table_t, nblk):
    vc = _REPACK_VC
    return pl.pallas_call(
        _repack_body,
        grid=(nblk,),
        in_specs=[pl.BlockSpec((EMBED_DIM, vc), lambda i: (0, i))],
        out_specs=pl.BlockSpec((vc // 4, 2 * EMBED_DIM), lambda i: (i, 0)),
        out_shape=jax.ShapeDtypeStruct((nblk * (vc // 4), 2 * EMBED_DIM),
                                       jnp.uint32),
    )(table_t)


def _extract(vec, j):
    lane = lax.broadcasted_iota(jnp.int32, (_LANES,), 0)
    masked = jnp.where(lane == j, vec, jnp.int32(0))
    return jnp.max(masked)


def _gather_body(lt_hbm, ct_hbm, li_hbm, ci_hbm, el_hbm, ec_hbm,
                 li_v, ci_v, el_v, ec_v, sem):
    wid = lax.axis_index("s") * _NC + lax.axis_index("c")
    base = wid * _B_PER_W
    pltpu.sync_copy(li_hbm.at[pl.ds(base, _B_PER_W)], li_v)
    pltpu.sync_copy(ci_hbm.at[pl.ds(base, _B_PER_W)], ci_v)
    for c in range(_NCHUNK):
        off = c * _CHUNK

        def issue(g, _):
            lv = li_v[pl.ds(off + g * _LANES, _LANES)]
            cv = ci_v[pl.ds(off + g * _LANES, _LANES)]
            for j in range(_LANES):
                li = _extract(lv, j)
                ci = _extract(cv, j)
                i = g * _LANES + j
                pltpu.make_async_copy(lt_hbm.at[pl.ds(li, 1), :],
                                      el_v.at[pl.ds(i, 1), :], sem).start()
                pltpu.make_async_copy(ct_hbm.at[pl.ds(ci, 1), :],
                                      ec_v.at[pl.ds(i, 1), :], sem).start()
            return 0

        lax.fori_loop(0, _CHUNK // _LANES, issue, 0)
        pltpu.make_async_copy(lt_hbm.at[pl.ds(0, _CHUNK), :], el_v, sem).wait()
        pltpu.make_async_copy(ct_hbm.at[pl.ds(0, _CHUNK), :], ec_v, sem).wait()
        pltpu.sync_copy(el_v, el_hbm.at[pl.ds(base + off, _CHUNK)])
        pltpu.sync_copy(ec_v, ec_hbm.at[pl.ds(base + off, _CHUNK)])


@jax.jit
def _sc_gather(lt_packed, ct_packed, li_half, ci_half):
    mesh = plsc.VectorSubcoreMesh(core_axis_name="c", subcore_axis_name="s")
    out_type = [
        jax.ShapeDtypeStruct((BATCH, _PACK), jnp.uint32),
        jax.ShapeDtypeStruct((BATCH, _PACK), jnp.uint32),
    ]
    scratch = [
        pltpu.VMEM((_B_PER_W,), jnp.int32),
        pltpu.VMEM((_B_PER_W,), jnp.int32),
        pltpu.VMEM((_CHUNK, _PACK), jnp.uint32),
        pltpu.VMEM((_CHUNK, _PACK), jnp.uint32),
        pltpu.SemaphoreType.DMA,
    ]
    fn = pl.kernel(_gather_body, out_type=out_type, mesh=mesh,
                   scratch_types=scratch,
                   compiler_params=pltpu.CompilerParams(
                       needs_layout_passes=False))
    return fn(lt_packed, ct_packed, li_half, ci_half)


def _unpack_select(xu, p):
    half = jnp.where(p >= 2, xu[:, EMBED_DIM:], xu[:, :EMBED_DIM])
    bits = jnp.where((p & 1) == 1, half >> 16, half & 0xFFFF)
    return lax.bitcast_convert_type(bits << 16, jnp.float32)


def _mlp_body(x1_ref, x2_ref, p1_ref, p2_ref, w1a_ref, w1b_ref, b1_ref,
              w2_ref, b2_ref, w3_ref, b3_ref, o_ref):
    x1 = _unpack_select(x1_ref[...], p1_ref[...])
    x2 = _unpack_select(x2_ref[...], p2_ref[...])
    h = jnp.dot(x1, w1a_ref[...], preferred_element_type=jnp.float32)
    h += jnp.dot(x2, w1b_ref[...], preferred_element_type=jnp.float32)
    h = jnp.maximum(h + b1_ref[...], 0.0)
    h = jnp.dot(h, w2_ref[...], preferred_element_type=jnp.float32)
    h = jnp.maximum(h + b2_ref[...], 0.0)
    o_ref[...] = jnp.dot(h, w3_ref[...],
                         preferred_element_type=jnp.float32) + b3_ref[...]


@functools.partial(jax.jit, static_argnames=("bm",))
def _tc_mlp(e_label, e_cat, p1, p2, W1a, W1b, b1, W2, b2, W3, b3, bm=2048):
    grid = (BATCH // bm,)
    full = lambda shape: pl.BlockSpec(shape, lambda i: (0, 0))
    return pl.pallas_call(
        _mlp_body,
        grid=grid,
        in_specs=[
            pl.BlockSpec((bm, _PACK), lambda i: (i, 0)),
            pl.BlockSpec((bm, _PACK), lambda i: (i, 0)),
            pl.BlockSpec((bm, 1), lambda i: (i, 0)),
            pl.BlockSpec((bm, 1), lambda i: (i, 0)),
            full(W1a.shape),
            full(W1b.shape),
            full(b1.shape),
            full(W2.shape),
            full(b2.shape),
            full(W3.shape),
            full(b3.shape),
        ],
        out_specs=pl.BlockSpec((bm, 2), lambda i: (i, 0)),
        out_shape=jax.ShapeDtypeStruct((BATCH, 2), jnp.float32),
    )(e_label, e_cat, p1, p2, W1a, W1b, b1, W2, b2, W3, b3)


def kernel(label_idx, category_idx, label_table, cat_table,
           W1, b1, W2, b2, W3, b3):
    li = label_idx.astype(jnp.int32)
    ci = category_idx.astype(jnp.int32)
    vc = _REPACK_VC
    nblk_l = -(-1000000 // vc)
    nblk_c = -(-100000 // vc)
    lt_packed = _tc_repack(label_table.T, nblk_l)
    ct_packed = _tc_repack(cat_table.T, nblk_c)
    qm = vc // 4 - 1
    lrow = ((li >> 15) << 13) + (li & qm)
    crow = ((ci >> 15) << 13) + (ci & qm)
    e_label, e_cat = _sc_gather(lt_packed, ct_packed, lrow, crow)
    p1 = ((li >> 13) & 3).reshape(-1, 1)
    p2 = ((ci >> 13) & 3).reshape(-1, 1)
    W1a = W1[:EMBED_DIM]
    W1b = W1[EMBED_DIM:]
    return _tc_mlp(e_label, e_cat, p1, p2, W1a, W1b, b1.reshape(1, -1), W2,
                   b2.reshape(1, -1), W3, b3.reshape(1, -1))


# speedup vs baseline: 1.1566x; 1.0165x over previous
import functools

import jax
import jax.numpy as jnp
from jax import lax
from jax.experimental import pallas as pl
from jax.experimental.pallas import tpu as pltpu
from jax.experimental.pallas import tpu_sc as plsc

BATCH = 16384
EMBED_DIM = 64
_PACK = 2 * EMBED_DIM

_NC, _NS = 2, 16
_NW = _NC * _NS
_B_PER_W = BATCH // _NW
_LANES = 16
_CHUNK = 256
_NCHUNK = _B_PER_W // _CHUNK


_REPACK_VC = 32768


def _bf16_bits(y):
    u = lax.bitcast_convert_type(y, jnp.uint32)
    return (u + 0x7FFF + ((u >> 16) & 1)) >> 16


def _repack_body(x_ref, o_ref):
    q = _REPACK_VC // 4
    x = x_ref[...]
    a = [_bf16_bits(x[:, k * q:(k + 1) * q].T) for k in range(4)]
    lo = a[0] | (a[1] << 16)
    hi = a[2] | (a[3] << 16)
    o_ref[...] = jnp.concatenate([lo, hi], axis=1)


@functools.partial(jax.jit, static_argnames=("nblk",))
def _tc_repack(table_t, nblk):
    vc = _REPACK_VC
    return pl.pallas_call(
        _repack_body,
        grid=(nblk,),
        in_specs=[pl.BlockSpec((EMBED_DIM, vc), lambda i: (0, i))],
        out_specs=pl.BlockSpec((vc // 4, 2 * EMBED_DIM), lambda i: (i, 0)),
        out_shape=jax.ShapeDtypeStruct((nblk * (vc // 4), 2 * EMBED_DIM),
                                       jnp.uint32),
    )(table_t)


def _extract(vec, j):
    lane = lax.broadcasted_iota(jnp.int32, (_LANES,), 0)
    masked = jnp.where(lane == j, vec, jnp.int32(0))
    return jnp.max(masked)


def _gather_body(t_hbm, i_hbm, o_hbm, i_v, o_v, sem):
    wid = lax.axis_index("s") * _NC + lax.axis_index("c")
    base = wid * _B_PER_W
    pltpu.sync_copy(i_hbm.at[pl.ds(base, _B_PER_W)], i_v)
    for c in range(_NCHUNK):
        off = c * _CHUNK

        def issue(g, _):
            iv = i_v[pl.ds(off + g * _LANES, _LANES)]
            for j in range(_LANES):
                r = _extract(iv, j)
                i = g * _LANES + j
                pltpu.make_async_copy(t_hbm.at[pl.ds(r, 1), :],
                                      o_v.at[pl.ds(i, 1), :], sem).start()
            return 0

        lax.fori_loop(0, _CHUNK // _LANES, issue, 0)
        pltpu.make_async_copy(t_hbm.at[pl.ds(0, _CHUNK), :], o_v, sem).wait()
        pltpu.sync_copy(o_v, o_hbm.at[pl.ds(base + off, _CHUNK)])


@jax.jit
def _sc_gather(t_packed, rows):
    mesh = plsc.VectorSubcoreMesh(core_axis_name="c", subcore_axis_name="s")
    out_type = jax.ShapeDtypeStruct((BATCH, _PACK), jnp.uint32)
    scratch = [
        pltpu.VMEM((_B_PER_W,), jnp.int32),
        pltpu.VMEM((_CHUNK, _PACK), jnp.uint32),
        pltpu.SemaphoreType.DMA,
    ]
    fn = pl.kernel(_gather_body, out_type=out_type, mesh=mesh,
                   scratch_types=scratch,
                   compiler_params=pltpu.CompilerParams(
                       needs_layout_passes=False))
    return fn(t_packed, rows)


def _unpack_select(xu, p):
    half = jnp.where(p >= 2, xu[:, EMBED_DIM:], xu[:, :EMBED_DIM])
    bits = jnp.where((p & 1) == 1, half >> 16, half & 0xFFFF)
    return lax.bitcast_convert_type(bits << 16, jnp.float32)


def _mlp_body(x1_ref, x2_ref, p1_ref, p2_ref, w1a_ref, w1b_ref, b1_ref,
              w2_ref, b2_ref, w3_ref, b3_ref, o_ref):
    x1 = _unpack_select(x1_ref[...], p1_ref[...])
    x2 = _unpack_select(x2_ref[...], p2_ref[...])
    h = jnp.dot(x1, w1a_ref[...], preferred_element_type=jnp.float32)
    h += jnp.dot(x2, w1b_ref[...], preferred_element_type=jnp.float32)
    h = jnp.maximum(h + b1_ref[...], 0.0)
    h = jnp.dot(h, w2_ref[...], preferred_element_type=jnp.float32)
    h = jnp.maximum(h + b2_ref[...], 0.0)
    o_ref[...] = jnp.dot(h, w3_ref[...],
                         preferred_element_type=jnp.float32) + b3_ref[...]


@functools.partial(jax.jit, static_argnames=("bm",))
def _tc_mlp(e_label, e_cat, p1, p2, W1a, W1b, b1, W2, b2, W3, b3, bm=2048):
    grid = (BATCH // bm,)
    full = lambda shape: pl.BlockSpec(shape, lambda i: (0, 0))
    return pl.pallas_call(
        _mlp_body,
        grid=grid,
        in_specs=[
            pl.BlockSpec((bm, _PACK), lambda i: (i, 0)),
            pl.BlockSpec((bm, _PACK), lambda i: (i, 0)),
            pl.BlockSpec((bm, 1), lambda i: (i, 0)),
            pl.BlockSpec((bm, 1), lambda i: (i, 0)),
            full(W1a.shape),
            full(W1b.shape),
            full(b1.shape),
            full(W2.shape),
            full(b2.shape),
            full(W3.shape),
            full(b3.shape),
        ],
        out_specs=pl.BlockSpec((bm, 2), lambda i: (i, 0)),
        out_shape=jax.ShapeDtypeStruct((BATCH, 2), jnp.float32),
    )(e_label, e_cat, p1, p2, W1a, W1b, b1, W2, b2, W3, b3)


def kernel(label_idx, category_idx, label_table, cat_table,
           W1, b1, W2, b2, W3, b3):
    li = label_idx.astype(jnp.int32)
    ci = category_idx.astype(jnp.int32)
    vc = _REPACK_VC
    nblk_l = -(-1000000 // vc)
    nblk_c = -(-100000 // vc)
    qm = vc // 4 - 1
    lrow = ((li >> 15) << 13) + (li & qm)
    crow = ((ci >> 15) << 13) + (ci & qm)
    ct_packed = _tc_repack(cat_table.T, nblk_c)
    e_cat = _sc_gather(ct_packed, crow)
    lt_packed = _tc_repack(label_table.T, nblk_l)
    e_label = _sc_gather(lt_packed, lrow)
    p1 = ((li >> 13) & 3).reshape(-1, 1)
    p2 = ((ci >> 13) & 3).reshape(-1, 1)
    W1a = W1[:EMBED_DIM]
    W1b = W1[EMBED_DIM:]
    return _tc_mlp(e_label, e_cat, p1, p2, W1a, W1b, b1.reshape(1, -1), W2,
                   b2.reshape(1, -1), W3, b3.reshape(1, -1))
